# Initial kernel scaffold; baseline (speedup 1.0000x reference)
import jax
import jax.numpy as jnp
from jax.experimental import pallas as pl


def kernel(node_feat, edge_feat, graph_dst, lg_src, lg_dst, three_basis, three_cutoff, segment_ids, W_atom, b_atom, W1, b1, Wg, bg):
    raise NotImplementedError("write your pallas kernel here")


# speedup vs baseline: 23.9694x; 23.9694x over previous
import jax
import jax.numpy as jnp
from jax import lax
from jax.experimental import pallas as pl
from jax.experimental.pallas import tpu as pltpu
from jax.experimental.pallas import tpu_sc as plsc

N, E, L, D, B = 10000, 320000, 3200000, 128, 9
BPAD = 16
NC, NS = 2, 16
NW = NC * NS
SEG_PER = E // NW
CHUNK = 1024
EDGE_CHUNK = 1000


def _atoms_body(x_ref, w_ref, b_ref, o_ref):
    x = jnp.dot(x_ref[...], w_ref[...], preferred_element_type=jnp.float32)
    o_ref[...] = jax.nn.sigmoid(x + b_ref[...][0:1, :])


def _compute_atoms(node_feat, w16, b16):
    blk = 1000
    return pl.pallas_call(
        _atoms_body,
        grid=(N // blk,),
        in_specs=[
            pl.BlockSpec((blk, D), lambda i: (i, 0)),
            pl.BlockSpec((D, BPAD), lambda i: (0, 0)),
            pl.BlockSpec((8, BPAD), lambda i: (0, 0)),
        ],
        out_specs=pl.BlockSpec((blk, BPAD), lambda i: (i, 0)),
        out_shape=jax.ShapeDtypeStruct((N, BPAD), jnp.float32),
    )(node_feat, w16, b16)


def _edge_gather_body(atoms_hbm, gdst_hbm, out_hbm, idx_v, rows_v, sem):
    c = lax.axis_index("c")
    s = lax.axis_index("s")
    w = s * NC + c
    base = w * (E // NW)
    for i in range(E // NW // EDGE_CHUNK):
        off = base + i * EDGE_CHUNK
        pltpu.sync_copy(gdst_hbm.at[pl.ds(off, EDGE_CHUNK)], idx_v)
        pltpu.async_copy(atoms_hbm.at[idx_v], rows_v, sem).wait()
        pltpu.sync_copy(rows_v, out_hbm.at[pl.ds(off, EDGE_CHUNK)])


def _edge_gather(atoms16, graph_dst):
    return pl.kernel(
        _edge_gather_body,
        out_type=jax.ShapeDtypeStruct((E, BPAD), jnp.float32),
        mesh=plsc.VectorSubcoreMesh(core_axis_name="c", subcore_axis_name="s"),
        scratch_types=[
            pltpu.VMEM((EDGE_CHUNK,), jnp.int32),
            pltpu.VMEM((EDGE_CHUNK, BPAD), jnp.float32),
            pltpu.SemaphoreType.DMA,
        ],
        compiler_params=pltpu.CompilerParams(use_tc_tiling_on_sc=False),
    )(atoms16, graph_dst)


def _segsum_body(tb_hbm, lg_hbm, seg_hbm, ea_hbm, cuts_hbm, out_hbm,
                 acc_v, tb_v, ea_v, lg_v, seg_v, cuts_v, sem):
    c = lax.axis_index("c")
    s = lax.axis_index("s")
    w = s * NC + c
    pltpu.sync_copy(cuts_hbm, cuts_v)
    lo = jnp.max(plsc.load_gather(cuts_v, [jnp.full((16,), w, jnp.int32)]))
    hi = jnp.max(plsc.load_gather(cuts_v, [jnp.full((16,), w + 1, jnp.int32)]))
    seg_base = w * SEG_PER

    zeros16 = jnp.zeros((16,), jnp.float32)

    def zero_body(i, _):
        acc_v[pl.ds(i * 16, 16)] = zeros16
        return 0

    lax.fori_loop(0, SEG_PER * B // 16, zero_body, 0)

    iota16 = lax.iota(jnp.int32, 16)

    def chunk_body(k, _):
        off = k * CHUNK
        pltpu.sync_copy(lg_hbm.at[pl.ds(off, CHUNK)], lg_v)
        gather = pltpu.async_copy(ea_hbm.at[lg_v], ea_v, sem)
        pltpu.sync_copy(seg_hbm.at[pl.ds(off, CHUNK)], seg_v)
        pltpu.sync_copy(tb_hbm.at[pl.ds(off * B, CHUNK * B)], tb_v)
        gather.wait()

        def group_body(g, _):
            rows = g * 16 + iota16
            seg16 = seg_v[pl.ds(g * 16, 16)]
            absi = off + rows
            m = (absi >= lo) & (absi < hi)
            local = jnp.clip(seg16 - seg_base, 0, SEG_PER - 1)
            for j in range(B):
                tbj = plsc.load_gather(tb_v, [rows * B + j])
                eaj = plsc.load_gather(
                    ea_v, [rows, jnp.full((16,), j, jnp.int32)])
                prod = jnp.where(m, tbj * eaj, 0.0)
                plsc.addupdate_scatter(acc_v, [local * B + j], prod)
            return 0

        lax.fori_loop(0, CHUNK // 16, group_body, 0)
        return 0

    lax.fori_loop(lo // CHUNK, (hi + CHUNK - 1) // CHUNK, chunk_body, 0)
    pltpu.sync_copy(acc_v, out_hbm.at[pl.ds(w * SEG_PER * B, SEG_PER * B)])


def _segsum(tb_flat, lg_dst, segment_ids, edge_atoms, cuts):
    return pl.kernel(
        _segsum_body,
        out_type=jax.ShapeDtypeStruct((E * B,), jnp.float32),
        mesh=plsc.VectorSubcoreMesh(core_axis_name="c", subcore_axis_name="s"),
        scratch_types=[
            pltpu.VMEM((SEG_PER * B,), jnp.float32),
            pltpu.VMEM((CHUNK * B,), jnp.float32),
            pltpu.VMEM((CHUNK, BPAD), jnp.float32),
            pltpu.VMEM((CHUNK,), jnp.int32),
            pltpu.VMEM((CHUNK,), jnp.int32),
            pltpu.VMEM((40,), jnp.int32),
            pltpu.SemaphoreType.DMA,
        ],
        compiler_params=pltpu.CompilerParams(use_tc_tiling_on_sc=False,
                                             needs_layout_passes=False),
    )(tb_flat, lg_dst, segment_ids, edge_atoms, cuts)


def _mlp_body(nb_ref, ef_ref, w1_ref, b1_ref, wg_ref, bg_ref, o_ref):
    nb = nb_ref[...]
    x = jnp.dot(nb, w1_ref[...], preferred_element_type=jnp.float32)
    x = x + b1_ref[...][0:1, :]
    g = jnp.dot(nb, wg_ref[...], preferred_element_type=jnp.float32)
    g = g + bg_ref[...][0:1, :]
    o_ref[...] = ef_ref[...] + (x * jax.nn.sigmoid(x)) * jax.nn.sigmoid(g)


def _mlp(nb16, edge_feat, w1p, b1t, wgp, bgt):
    blk = 1280
    return pl.pallas_call(
        _mlp_body,
        grid=(E // blk,),
        in_specs=[
            pl.BlockSpec((blk, BPAD), lambda i: (i, 0)),
            pl.BlockSpec((blk, D), lambda i: (i, 0)),
            pl.BlockSpec((BPAD, D), lambda i: (0, 0)),
            pl.BlockSpec((8, D), lambda i: (0, 0)),
            pl.BlockSpec((BPAD, D), lambda i: (0, 0)),
            pl.BlockSpec((8, D), lambda i: (0, 0)),
        ],
        out_specs=pl.BlockSpec((blk, D), lambda i: (i, 0)),
        out_shape=jax.ShapeDtypeStruct((E, D), jnp.float32),
    )(nb16, edge_feat, w1p, b1t, wgp, bgt)


def kernel(node_feat, edge_feat, graph_dst, lg_src, lg_dst, three_basis,
           three_cutoff, segment_ids, W_atom, b_atom, W1, b1, Wg, bg):
    w16 = jnp.pad(W_atom, ((0, 0), (0, BPAD - B)))
    b16 = jnp.tile(jnp.pad(b_atom, (0, BPAD - B))[None, :], (8, 1))
    atoms16 = _compute_atoms(node_feat, w16, b16)

    edge_atoms = _edge_gather(atoms16, graph_dst.astype(jnp.int32))

    cuts = jnp.searchsorted(
        segment_ids, jnp.arange(0, E + 1, SEG_PER)).astype(jnp.int32)
    cuts = jnp.pad(cuts, (0, 7))
    nb_flat = _segsum(three_basis.reshape(L * B),
                      lg_dst.astype(jnp.int32),
                      segment_ids.astype(jnp.int32),
                      edge_atoms, cuts)

    nb16 = jnp.pad(nb_flat.reshape(E, B), ((0, 0), (0, BPAD - B)))
    w1p = jnp.pad(W1, ((0, BPAD - B), (0, 0)))
    wgp = jnp.pad(Wg, ((0, BPAD - B), (0, 0)))
    b1t = jnp.tile(b1[None, :], (8, 1))
    bgt = jnp.tile(bg[None, :], (8, 1))
    return _mlp(nb16, edge_feat, w1p, b1t, wgp, bgt)


# speedup vs baseline: 27.4917x; 1.1469x over previous
import jax
import jax.numpy as jnp
from jax import lax
from jax.experimental import pallas as pl
from jax.experimental.pallas import tpu as pltpu
from jax.experimental.pallas import tpu_sc as plsc

N, E, L, D, B = 10000, 320000, 3200000, 128, 9
BPAD = 16
NC, NS = 2, 16
NW = NC * NS
SEG_PER = E // NW
CHUNK = 1024
EDGE_CHUNK = 1000


def _atoms_body(x_ref, w_ref, b_ref, o_ref):
    x = jnp.dot(x_ref[...], w_ref[...], preferred_element_type=jnp.float32)
    o_ref[...] = jax.nn.sigmoid(x + b_ref[...][0:1, :])


def _compute_atoms(node_feat, w16, b16):
    blk = 1000
    return pl.pallas_call(
        _atoms_body,
        grid=(N // blk,),
        in_specs=[
            pl.BlockSpec((blk, D), lambda i: (i, 0)),
            pl.BlockSpec((D, BPAD), lambda i: (0, 0)),
            pl.BlockSpec((8, BPAD), lambda i: (0, 0)),
        ],
        out_specs=pl.BlockSpec((blk, BPAD), lambda i: (i, 0)),
        out_shape=jax.ShapeDtypeStruct((N, BPAD), jnp.float32),
    )(node_feat, w16, b16)


def _edge_gather_body(atoms_hbm, gdst_hbm, out_hbm, idx_v, rows_v, sem):
    c = lax.axis_index("c")
    s = lax.axis_index("s")
    w = s * NC + c
    base = w * (E // NW)
    for i in range(E // NW // EDGE_CHUNK):
        off = base + i * EDGE_CHUNK
        pltpu.sync_copy(gdst_hbm.at[pl.ds(off, EDGE_CHUNK)], idx_v)
        pltpu.async_copy(atoms_hbm.at[idx_v], rows_v, sem).wait()
        pltpu.sync_copy(rows_v, out_hbm.at[pl.ds(off, EDGE_CHUNK)])


def _edge_gather(atoms16, graph_dst):
    return pl.kernel(
        _edge_gather_body,
        out_type=jax.ShapeDtypeStruct((E, BPAD), jnp.float32),
        mesh=plsc.VectorSubcoreMesh(core_axis_name="c", subcore_axis_name="s"),
        scratch_types=[
            pltpu.VMEM((EDGE_CHUNK,), jnp.int32),
            pltpu.VMEM((EDGE_CHUNK, BPAD), jnp.float32),
            pltpu.SemaphoreType.DMA,
        ],
        compiler_params=pltpu.CompilerParams(use_tc_tiling_on_sc=False),
    )(atoms16, graph_dst)


def _segsum_body(tb_hbm, lg_hbm, seg_hbm, ea_hbm, cuts_hbm, out_hbm,
                 acc_v, tb_v, ea_v, lg_v, seg_v, cuts_v, sem):
    c = lax.axis_index("c")
    s = lax.axis_index("s")
    w = s * NC + c
    pltpu.sync_copy(cuts_hbm, cuts_v)
    lo = jnp.max(plsc.load_gather(cuts_v, [jnp.full((16,), w, jnp.int32)]))
    hi = jnp.max(plsc.load_gather(cuts_v, [jnp.full((16,), w + 1, jnp.int32)]))
    seg_base = w * SEG_PER

    zeros16 = jnp.zeros((16,), jnp.float32)

    @plsc.parallel_loop(0, SEG_PER * B // 16, unroll=8)
    def _(i):
        acc_v[pl.ds(i * 16, 16)] = zeros16

    iota16 = lax.iota(jnp.int32, 16)

    def chunk_body(k, _):
        off = k * CHUNK
        pltpu.sync_copy(lg_hbm.at[pl.ds(off, CHUNK)], lg_v)
        gather = pltpu.async_copy(ea_hbm.at[lg_v], ea_v, sem)
        pltpu.sync_copy(seg_hbm.at[pl.ds(off, CHUNK)], seg_v)
        pltpu.sync_copy(tb_hbm.at[pl.ds(off * B, CHUNK * B)], tb_v)
        gather.wait()

        @plsc.parallel_loop(0, CHUNK // 16, unroll=2)
        def _(g):
            rows = g * 16 + iota16
            seg16 = seg_v[pl.ds(g * 16, 16)]
            absi = off + rows
            m = (absi >= lo) & (absi < hi)
            local = jnp.clip(seg16 - seg_base, 0, SEG_PER - 1)
            for j in range(B):
                tbj = plsc.load_gather(tb_v, [rows * B + j])
                eaj = plsc.load_gather(
                    ea_v, [rows, jnp.full((16,), j, jnp.int32)])
                prod = jnp.where(m, tbj * eaj, 0.0)
                plsc.addupdate_scatter(acc_v, [local * B + j], prod)

        return 0

    lax.fori_loop(lo // CHUNK, (hi + CHUNK - 1) // CHUNK, chunk_body, 0)
    pltpu.sync_copy(acc_v, out_hbm.at[pl.ds(w * SEG_PER * B, SEG_PER * B)])


def _segsum(tb_flat, lg_dst, segment_ids, edge_atoms, cuts):
    return pl.kernel(
        _segsum_body,
        out_type=jax.ShapeDtypeStruct((E * B,), jnp.float32),
        mesh=plsc.VectorSubcoreMesh(core_axis_name="c", subcore_axis_name="s"),
        scratch_types=[
            pltpu.VMEM((SEG_PER * B,), jnp.float32),
            pltpu.VMEM((CHUNK * B,), jnp.float32),
            pltpu.VMEM((CHUNK, BPAD), jnp.float32),
            pltpu.VMEM((CHUNK,), jnp.int32),
            pltpu.VMEM((CHUNK,), jnp.int32),
            pltpu.VMEM((40,), jnp.int32),
            pltpu.SemaphoreType.DMA,
        ],
        compiler_params=pltpu.CompilerParams(use_tc_tiling_on_sc=False,
                                             needs_layout_passes=False),
    )(tb_flat, lg_dst, segment_ids, edge_atoms, cuts)


def _mlp_body(nb_ref, ef_ref, w1_ref, b1_ref, wg_ref, bg_ref, o_ref):
    nb = nb_ref[...]
    x = jnp.dot(nb, w1_ref[...], preferred_element_type=jnp.float32)
    x = x + b1_ref[...][0:1, :]
    g = jnp.dot(nb, wg_ref[...], preferred_element_type=jnp.float32)
    g = g + bg_ref[...][0:1, :]
    o_ref[...] = ef_ref[...] + (x * jax.nn.sigmoid(x)) * jax.nn.sigmoid(g)


def _mlp(nb16, edge_feat, w1p, b1t, wgp, bgt):
    blk = 1280
    return pl.pallas_call(
        _mlp_body,
        grid=(E // blk,),
        in_specs=[
            pl.BlockSpec((blk, BPAD), lambda i: (i, 0)),
            pl.BlockSpec((blk, D), lambda i: (i, 0)),
            pl.BlockSpec((BPAD, D), lambda i: (0, 0)),
            pl.BlockSpec((8, D), lambda i: (0, 0)),
            pl.BlockSpec((BPAD, D), lambda i: (0, 0)),
            pl.BlockSpec((8, D), lambda i: (0, 0)),
        ],
        out_specs=pl.BlockSpec((blk, D), lambda i: (i, 0)),
        out_shape=jax.ShapeDtypeStruct((E, D), jnp.float32),
    )(nb16, edge_feat, w1p, b1t, wgp, bgt)


def kernel(node_feat, edge_feat, graph_dst, lg_src, lg_dst, three_basis,
           three_cutoff, segment_ids, W_atom, b_atom, W1, b1, Wg, bg):
    w16 = jnp.pad(W_atom, ((0, 0), (0, BPAD - B)))
    b16 = jnp.tile(jnp.pad(b_atom, (0, BPAD - B))[None, :], (8, 1))
    atoms16 = _compute_atoms(node_feat, w16, b16)

    edge_atoms = _edge_gather(atoms16, graph_dst.astype(jnp.int32))

    cuts = jnp.searchsorted(
        segment_ids, jnp.arange(0, E + 1, SEG_PER)).astype(jnp.int32)
    cuts = jnp.pad(cuts, (0, 7))
    nb_flat = _segsum(three_basis.reshape(L * B),
                      lg_dst.astype(jnp.int32),
                      segment_ids.astype(jnp.int32),
                      edge_atoms, cuts)

    nb16 = jnp.pad(nb_flat.reshape(E, B), ((0, 0), (0, BPAD - B)))
    w1p = jnp.pad(W1, ((0, BPAD - B), (0, 0)))
    wgp = jnp.pad(Wg, ((0, BPAD - B), (0, 0)))
    b1t = jnp.tile(b1[None, :], (8, 1))
    bgt = jnp.tile(bg[None, :], (8, 1))
    return _mlp(nb16, edge_feat, w1p, b1t, wgp, bgt)
